# speedup vs baseline: 15.9141x; 15.9141x over previous
import functools

import jax
import jax.numpy as jnp
from jax.experimental import pallas as pl


def _get_positive_k(k, n):
    if k <= 0:
        return 0
    elif k < 1:
        return round(k * n)
    elif k > n:
        return int(n)
    else:
        return int(k)


def _topk_mean_body(x_ref, o_ref, *, kmax, n_iters):
    x = x_ref[...]
    b = jax.lax.bitcast_convert_type(x, jnp.int32)
    m = jnp.where(b >= 0, b, -(b & jnp.int32(0x7FFFFFFF)))
    rows = x.shape[0]
    lo0 = jnp.full((rows, 1), -0x7F800001, dtype=jnp.int32)
    hi0 = jnp.full((rows, 1), 0x7F800001, dtype=jnp.int32)

    def body(_, carry):
        lo, hi = carry
        mid = (lo >> 1) + (hi >> 1) + (lo & hi & 1)
        cnt = jnp.sum((m > mid).astype(jnp.int32), axis=1, keepdims=True)
        pred = cnt >= kmax
        return jnp.where(pred, mid, lo), jnp.where(pred, hi, mid)

    lo, hi = jax.lax.fori_loop(0, n_iters, body, (lo0, hi0))
    t_m = hi
    gt = m > t_m
    cnt = jnp.sum(gt.astype(jnp.int32), axis=1, keepdims=True)
    sum_gt = jnp.sum(jnp.where(gt, x, 0.0), axis=1, keepdims=True)
    t_b = jnp.where(t_m >= 0, t_m, (-t_m) | jnp.int32(-0x80000000))
    t_f = jax.lax.bitcast_convert_type(t_b, jnp.float32)
    o_ref[...] = (sum_gt + (kmax - cnt).astype(jnp.float32) * t_f) * (
        1.0 / kmax
    )


def kernel(input):
    batch, chan, h, w = input.shape
    n = h * w
    kmax = _get_positive_k(0.2, n)
    rows = batch * chan
    x = input.reshape(rows, n)
    block_rows = 8
    grid = rows // block_rows
    out = pl.pallas_call(
        functools.partial(_topk_mean_body, kmax=kmax, n_iters=34),
        grid=(grid,),
        in_specs=[pl.BlockSpec((block_rows, n), lambda i: (i, 0))],
        out_specs=pl.BlockSpec((block_rows, 1), lambda i: (i, 0)),
        out_shape=jax.ShapeDtypeStruct((rows, 1), jnp.float32),
    )(x)
    return out.reshape(batch, chan)
